# Initial kernel scaffold; baseline (speedup 1.0000x reference)
import jax
import jax.numpy as jnp
from jax.experimental import pallas as pl


def kernel(data, service_matrix, embedding_matrix):
    raise NotImplementedError("write your pallas kernel here")


# speedup vs baseline: 4.8124x; 4.8124x over previous
import functools

import jax
import jax.numpy as jnp
from jax import lax
from jax.experimental import pallas as pl
from jax.experimental.pallas import tpu as pltpu
from jax.experimental.pallas import tpu_sc as plsc

NUM_SERVICES = 100000
ENC = 512
EMB = 64
BATCH = 16384

NC = 2
NS = 16
NW = NC * NS
B_PER_W = BATCH // NW
CHUNK = 64
N_CHUNKS = B_PER_W // CHUNK


def _make_sc_gather():
    mesh = plsc.VectorSubcoreMesh(core_axis_name="c", subcore_axis_name="s")

    @functools.partial(
        pl.kernel,
        mesh=mesh,
        out_type=jax.ShapeDtypeStruct((BATCH, ENC), jnp.float32),
        scratch_types=[
            pltpu.VMEM((B_PER_W,), jnp.int32),
            pltpu.VMEM((CHUNK, ENC), jnp.float32),
            pltpu.VMEM((CHUNK, ENC), jnp.float32),
            pltpu.SemaphoreType.DMA,
            pltpu.SemaphoreType.DMA,
        ],
    )
    def gather_k(idx_hbm, table_hbm, out_hbm, idx_v, rows0, rows1, sem0, sem1):
        wid = lax.axis_index("s") * NC + lax.axis_index("c")
        base = wid * B_PER_W
        pltpu.sync_copy(idx_hbm.at[pl.ds(base, B_PER_W)], idx_v)
        bufs = (rows0, rows1)
        sems = (sem0, sem1)
        handles = [None, None]
        handles[0] = pltpu.async_copy(
            table_hbm.at[idx_v.at[pl.ds(0, CHUNK)]], rows0, sem0)
        for ci in range(N_CHUNKS):
            if ci + 1 < N_CHUNKS:
                handles[(ci + 1) % 2] = pltpu.async_copy(
                    table_hbm.at[idx_v.at[pl.ds((ci + 1) * CHUNK, CHUNK)]],
                    bufs[(ci + 1) % 2],
                    sems[(ci + 1) % 2],
                )
            handles[ci % 2].wait()
            pltpu.sync_copy(bufs[ci % 2], out_hbm.at[pl.ds(base + ci * CHUNK, CHUNK)])

    return gather_k


_sc_gather = _make_sc_gather()


def _mm_body(s_ref, e_ref, o_ref):
    o_ref[...] = jnp.dot(s_ref[...], e_ref[...], preferred_element_type=jnp.float32)


def kernel(data, service_matrix, embedding_matrix):
    gathered = _sc_gather(data, service_matrix)
    out = pl.pallas_call(
        _mm_body,
        grid=(8,),
        in_specs=[
            pl.BlockSpec((BATCH // 8, ENC), lambda i: (i, 0)),
            pl.BlockSpec((ENC, EMB), lambda i: (0, 0)),
        ],
        out_specs=pl.BlockSpec((BATCH // 8, EMB), lambda i: (i, 0)),
        out_shape=jax.ShapeDtypeStruct((BATCH, EMB), jnp.float32),
    )(gathered, embedding_matrix)
    return out
